# baseline (speedup 1.0000x reference)
import jax
import jax.numpy as jnp
from jax import lax
from jax.experimental import pallas as pl
from jax.experimental.pallas import tpu as pltpu
from jax.experimental.pallas import tpu_sc as plsc

B = 4096
L = 200
EMB = 64
V = 1000000
VW = V // 2
NW = 32
SCALE = 8.0

CHUNK_T = 128
WCH = CHUNK_T // 2
NFULL = V // CHUNK_T
TAIL_T = V - NFULL * CHUNK_T
BASE_CH = NFULL // NW
EXTRA_W = NFULL - BASE_CH * NW

BBLK = B // NW


def _iota16():
    return jax.lax.iota(jnp.int32, 16)


def _t_body(tableT_hbm, tail_hbm, wide_hbm,
            blk0, blk1, buf0, buf1, si0, si1, so0, so1):
    wid = lax.axis_index("s") * 2 + lax.axis_index("c")
    nch = BASE_CH + jnp.where(wid < EXTRA_W, 1, 0)
    lanes = _iota16()
    blks = (blk0, blk1)
    bufs = (buf0, buf1)
    sis = (si0, si1)
    sos = (so0, so1)

    def load(i, slot, sem):
        t0 = pl.multiple_of((wid + NW * i) * CHUNK_T, CHUNK_T)
        pltpu.async_copy(tableT_hbm.at[:, pl.ds(t0, CHUNK_T)],
                         slot.at[:, pl.ds(0, CHUNK_T)], sem)

    def wait_load(slot, sem):
        pltpu.make_async_copy(tableT_hbm.at[:, pl.ds(0, CHUNK_T)],
                              slot.at[:, pl.ds(0, CHUNK_T)], sem).wait()

    def wait_store(slot, sem):
        pltpu.make_async_copy(slot, wide_hbm.at[pl.ds(0, WCH)], sem).wait()

    def transpose(blk, buf):
        def qgroup(q2, carry):
            for qq in range(4):
                q = 4 * q2 + qq
                c0 = jnp.broadcast_to(2 * q, (16,))
                c1 = jnp.broadcast_to(2 * q + 1, (16,))
                for k in range(8):
                    jv = lanes + (16 * k) % 64
                    buf[q, pl.ds(16 * k, 16)] = plsc.load_gather(
                        blk, [jv, c1 if k >= 4 else c0])
            return carry
        lax.fori_loop(0, WCH // 4, qgroup, 0)

    def store(i, slot, sem):
        w0 = pl.multiple_of((wid + NW * i) * WCH, WCH)
        pltpu.async_copy(slot, wide_hbm.at[pl.ds(w0, WCH)], sem)

    load(0, blk0, si0)

    def outer(i2, carry):
        for b in range(2):
            i = 2 * i2 + b

            @pl.when(i + 1 < nch)
            def _pf():
                load(i + 1, blks[1 - b], sis[1 - b])

            wait_load(blks[b], sis[b])

            @pl.when(i >= 2)
            def _wd():
                wait_store(bufs[b], sos[b])

            transpose(blks[b], bufs[b])
            store(i, bufs[b], sos[b])
        return carry

    lax.fori_loop(0, BASE_CH // 2, outer, 0)

    @pl.when(nch % 2 == 1)
    def _last():
        i = nch - 1
        wait_load(blks[0], sis[0])
        wait_store(bufs[0], sos[0])
        transpose(blks[0], bufs[0])
        store(i, bufs[0], sos[0])

    @pl.when(wid == NW - 1)
    def _tail():
        pltpu.sync_copy(tail_hbm,
                        blk0.at[pl.ds(0, TAIL_T // 2), pl.ds(0, 128)])
        pltpu.sync_copy(blk0.at[pl.ds(0, TAIL_T // 2), pl.ds(0, 128)],
                        wide_hbm.at[pl.ds(VW - TAIL_T // 2, TAIL_T // 2)])

    for b in range(2):
        wait_store(bufs[b], sos[b])


def _g_body(tokT_hbm, wide_hbm, out_hbm, tok_v,
            idx0, idx1, h0, h1, rows0, rows1, buf0, buf1,
            st, sg0, sg1, sw0, sw1):
    wid = lax.axis_index("s") * 2 + lax.axis_index("c")
    b0 = pl.multiple_of(wid * BBLK, BBLK)
    lanes = _iota16()
    idxs = (idx0, idx1)
    hs = (h0, h1)
    rows = (rows0, rows1)
    bufs = (buf0, buf1)
    sgs = (sg0, sg1)
    sws = (sw0, sw1)

    pltpu.async_copy(tokT_hbm.at[:, pl.ds(b0, BBLK)], tok_v, st).wait()

    def prep_and_gather(l, slot):
        for k in range(BBLK // 16):
            t = tok_v[l, pl.ds(16 * k, 16)]
            idxs[slot][pl.ds(16 * k, 16)] = lax.shift_right_logical(t, 1)
            hs[slot][pl.ds(16 * k, 16)] = lax.shift_left(t & 1, 6)
        pltpu.async_copy(wide_hbm.at[idxs[slot]],
                         rows[slot].at[:, pl.ds(0, 128)], sgs[slot])

    def wait_gather(slot):
        pltpu.make_async_copy(wide_hbm.at[idxs[slot]],
                              rows[slot].at[:, pl.ds(0, 128)],
                              sgs[slot]).wait()

    def wait_store(slot):
        pltpu.make_async_copy(bufs[slot],
                              out_hbm.at[0, :, pl.ds(0, BBLK)],
                              sws[slot]).wait()

    def transpose(slot):
        rv = rows[slot]
        bv = bufs[slot]
        for m in range(BBLK // 16):
            rvec = lanes + 16 * m
            hm = hs[slot][pl.ds(16 * m, 16)]

            def jgroup(j2, carry):
                for jj in range(4):
                    j = 4 * j2 + jj
                    bv[j, pl.ds(16 * m, 16)] = (
                        plsc.load_gather(rv, [rvec, hm + j]) * SCALE)
                return carry
            lax.fori_loop(0, EMB // 4, jgroup, 0)

    prep_and_gather(0, 0)

    def outer(l2, carry):
        for b in range(2):
            l = 2 * l2 + b

            @pl.when(l + 1 < L)
            def _pf():
                prep_and_gather(l + 1, 1 - b)

            wait_gather(b)

            @pl.when(l >= 2)
            def _wd():
                wait_store(b)

            transpose(b)
            pltpu.async_copy(bufs[b], out_hbm.at[l, :, pl.ds(b0, BBLK)],
                             sws[b])
        return carry

    lax.fori_loop(0, L // 2, outer, 0)
    for b in range(2):
        wait_store(b)


def kernel(tokens, table):
    mesh = plsc.VectorSubcoreMesh(core_axis_name="c", subcore_axis_name="s")
    wide = pl.kernel(
        _t_body,
        out_type=jax.ShapeDtypeStruct((VW, 128), jnp.float32),
        mesh=mesh,
        scratch_types=[
            pltpu.VMEM((EMB, CHUNK_T + 1), jnp.float32),
            pltpu.VMEM((EMB, CHUNK_T + 1), jnp.float32),
            pltpu.VMEM((WCH, 128), jnp.float32),
            pltpu.VMEM((WCH, 128), jnp.float32),
            pltpu.SemaphoreType.DMA,
            pltpu.SemaphoreType.DMA,
            pltpu.SemaphoreType.DMA,
            pltpu.SemaphoreType.DMA,
        ],
        compiler_params=pltpu.CompilerParams(needs_layout_passes=False),
    )(table.T, table[NFULL * CHUNK_T:].reshape(TAIL_T // 2, 128))
    outT = pl.kernel(
        _g_body,
        out_type=jax.ShapeDtypeStruct((L, EMB, B), jnp.float32),
        mesh=mesh,
        scratch_types=[
            pltpu.VMEM((L, BBLK), jnp.int32),
            pltpu.VMEM((BBLK,), jnp.int32),
            pltpu.VMEM((BBLK,), jnp.int32),
            pltpu.VMEM((BBLK,), jnp.int32),
            pltpu.VMEM((BBLK,), jnp.int32),
            pltpu.VMEM((BBLK, 129), jnp.float32),
            pltpu.VMEM((BBLK, 129), jnp.float32),
            pltpu.VMEM((EMB, BBLK), jnp.float32),
            pltpu.VMEM((EMB, BBLK), jnp.float32),
            pltpu.SemaphoreType.DMA,
            pltpu.SemaphoreType.DMA,
            pltpu.SemaphoreType.DMA,
            pltpu.SemaphoreType.DMA,
            pltpu.SemaphoreType.DMA,
        ],
        compiler_params=pltpu.CompilerParams(needs_layout_passes=False),
    )(tokens.T.astype(jnp.int32), wide)
    return outT.transpose(2, 0, 1)


# speedup vs baseline: 1.5308x; 1.5308x over previous
import jax
import jax.numpy as jnp
from jax import lax
from jax.experimental import pallas as pl
from jax.experimental.pallas import tpu as pltpu
from jax.experimental.pallas import tpu_sc as plsc

B = 4096
L = 200
EMB = 64
V = 1000000
VW = V // 2
NW = 32
SCALE = 8.0

CHUNK_T = 128
WCH = CHUNK_T // 2
NFULL = V // CHUNK_T
TAIL_T = V - NFULL * CHUNK_T
BASE_CH = NFULL // NW
EXTRA_W = NFULL - BASE_CH * NW

BBLK = B // NW


def _iota16():
    return jax.lax.iota(jnp.int32, 16)


def _t_body(tableT_hbm, tail_hbm, wide_hbm,
            blk0, blk1, buf0, buf1, si0, si1, so0, so1):
    wid = lax.axis_index("s") * 2 + lax.axis_index("c")
    nch = BASE_CH + jnp.where(wid < EXTRA_W, 1, 0)
    lanes = _iota16()
    blks = (blk0, blk1)
    bufs = (buf0, buf1)
    sis = (si0, si1)
    sos = (so0, so1)

    def load(i, slot, sem):
        t0 = pl.multiple_of((wid + NW * i) * CHUNK_T, CHUNK_T)
        pltpu.async_copy(tableT_hbm.at[:, pl.ds(t0, CHUNK_T)],
                         slot.at[:, pl.ds(0, CHUNK_T)], sem)

    def wait_load(slot, sem):
        pltpu.make_async_copy(tableT_hbm.at[:, pl.ds(0, CHUNK_T)],
                              slot.at[:, pl.ds(0, CHUNK_T)], sem).wait()

    def wait_store(slot, sem):
        pltpu.make_async_copy(slot, wide_hbm.at[pl.ds(0, WCH)], sem).wait()

    def transpose(blk, buf):
        jvs = [lanes + 16 * k for k in range(4)]

        def qgroup(q2, carry):
            for qq in range(2):
                q = 2 * q2 + qq
                c0 = jnp.broadcast_to(2 * q, (16,))
                c1 = jnp.broadcast_to(2 * q + 1, (16,))
                vals = [plsc.load_gather(blk, [jvs[k % 4],
                                               c1 if k >= 4 else c0])
                        for k in range(8)]
                for k in range(8):
                    buf[q, pl.ds(16 * k, 16)] = vals[k]
            return carry
        lax.fori_loop(0, WCH // 2, qgroup, 0)

    def store(i, slot, sem):
        w0 = pl.multiple_of((wid + NW * i) * WCH, WCH)
        pltpu.async_copy(slot, wide_hbm.at[pl.ds(w0, WCH)], sem)

    load(0, blk0, si0)

    def outer(i2, carry):
        for b in range(2):
            i = 2 * i2 + b

            @pl.when(i + 1 < nch)
            def _pf():
                load(i + 1, blks[1 - b], sis[1 - b])

            wait_load(blks[b], sis[b])

            @pl.when(i >= 2)
            def _wd():
                wait_store(bufs[b], sos[b])

            transpose(blks[b], bufs[b])
            store(i, bufs[b], sos[b])
        return carry

    lax.fori_loop(0, BASE_CH // 2, outer, 0)

    @pl.when(nch % 2 == 1)
    def _last():
        i = nch - 1
        wait_load(blks[0], sis[0])
        wait_store(bufs[0], sos[0])
        transpose(blks[0], bufs[0])
        store(i, bufs[0], sos[0])

    @pl.when(wid == NW - 1)
    def _tail():
        pltpu.sync_copy(tail_hbm,
                        blk0.at[pl.ds(0, TAIL_T // 2), pl.ds(0, 128)])
        pltpu.sync_copy(blk0.at[pl.ds(0, TAIL_T // 2), pl.ds(0, 128)],
                        wide_hbm.at[pl.ds(VW - TAIL_T // 2, TAIL_T // 2)])

    for b in range(2):
        wait_store(bufs[b], sos[b])


def _g_body(tokT_hbm, wide_hbm, out_hbm, tok_v,
            idx0, idx1, h0, h1, rows0, rows1, buf0, buf1,
            st, sg0, sg1, sw0, sw1):
    wid = lax.axis_index("s") * 2 + lax.axis_index("c")
    b0 = pl.multiple_of(wid * BBLK, BBLK)
    lanes = _iota16()
    idxs = (idx0, idx1)
    hs = (h0, h1)
    rows = (rows0, rows1)
    bufs = (buf0, buf1)
    sgs = (sg0, sg1)
    sws = (sw0, sw1)

    pltpu.async_copy(tokT_hbm.at[:, pl.ds(b0, BBLK)], tok_v, st).wait()

    def prep_and_gather(l, slot):
        for k in range(BBLK // 16):
            t = tok_v[l, pl.ds(16 * k, 16)]
            idxs[slot][pl.ds(16 * k, 16)] = lax.shift_right_logical(t, 1)
            hs[slot][pl.ds(16 * k, 16)] = lax.shift_left(t & 1, 6)
        pltpu.async_copy(wide_hbm.at[idxs[slot]],
                         rows[slot].at[:, pl.ds(0, 128)], sgs[slot])

    def wait_gather(slot):
        pltpu.make_async_copy(wide_hbm.at[idxs[slot]],
                              rows[slot].at[:, pl.ds(0, 128)],
                              sgs[slot]).wait()

    def wait_store(slot):
        pltpu.make_async_copy(bufs[slot],
                              out_hbm.at[0, :, pl.ds(0, BBLK)],
                              sws[slot]).wait()

    def transpose(slot):
        rv = rows[slot]
        bv = bufs[slot]
        for m in range(BBLK // 16):
            rvec = lanes + 16 * m
            hm = hs[slot][pl.ds(16 * m, 16)]

            def jgroup(j2, carry):
                base = j2 * 8
                vals = [plsc.load_gather(rv, [rvec, hm + (base + jj)]) * SCALE
                        for jj in range(8)]
                for jj in range(8):
                    bv[base + jj, pl.ds(16 * m, 16)] = vals[jj]
                return carry
            lax.fori_loop(0, EMB // 8, jgroup, 0)

    prep_and_gather(0, 0)

    def outer(l2, carry):
        for b in range(2):
            l = 2 * l2 + b

            @pl.when(l + 1 < L)
            def _pf():
                prep_and_gather(l + 1, 1 - b)

            wait_gather(b)

            @pl.when(l >= 2)
            def _wd():
                wait_store(b)

            transpose(b)
            pltpu.async_copy(bufs[b], out_hbm.at[l, :, pl.ds(b0, BBLK)],
                             sws[b])
        return carry

    lax.fori_loop(0, L // 2, outer, 0)
    for b in range(2):
        wait_store(b)


def kernel(tokens, table):
    mesh = plsc.VectorSubcoreMesh(core_axis_name="c", subcore_axis_name="s")
    wide = pl.kernel(
        _t_body,
        out_type=jax.ShapeDtypeStruct((VW, 128), jnp.float32),
        mesh=mesh,
        scratch_types=[
            pltpu.VMEM((EMB, CHUNK_T + 1), jnp.float32),
            pltpu.VMEM((EMB, CHUNK_T + 1), jnp.float32),
            pltpu.VMEM((WCH, 128), jnp.float32),
            pltpu.VMEM((WCH, 128), jnp.float32),
            pltpu.SemaphoreType.DMA,
            pltpu.SemaphoreType.DMA,
            pltpu.SemaphoreType.DMA,
            pltpu.SemaphoreType.DMA,
        ],
        compiler_params=pltpu.CompilerParams(needs_layout_passes=False),
    )(table.T, table[NFULL * CHUNK_T:].reshape(TAIL_T // 2, 128))
    outT = pl.kernel(
        _g_body,
        out_type=jax.ShapeDtypeStruct((L, EMB, B), jnp.float32),
        mesh=mesh,
        scratch_types=[
            pltpu.VMEM((L, BBLK), jnp.int32),
            pltpu.VMEM((BBLK,), jnp.int32),
            pltpu.VMEM((BBLK,), jnp.int32),
            pltpu.VMEM((BBLK,), jnp.int32),
            pltpu.VMEM((BBLK,), jnp.int32),
            pltpu.VMEM((BBLK, 129), jnp.float32),
            pltpu.VMEM((BBLK, 129), jnp.float32),
            pltpu.VMEM((EMB, BBLK), jnp.float32),
            pltpu.VMEM((EMB, BBLK), jnp.float32),
            pltpu.SemaphoreType.DMA,
            pltpu.SemaphoreType.DMA,
            pltpu.SemaphoreType.DMA,
            pltpu.SemaphoreType.DMA,
            pltpu.SemaphoreType.DMA,
        ],
        compiler_params=pltpu.CompilerParams(needs_layout_passes=False),
    )(tokens.T.astype(jnp.int32), wide)
    return outT.transpose(2, 0, 1)


# speedup vs baseline: 1.5576x; 1.0175x over previous
import jax
import jax.numpy as jnp
from jax import lax
from jax.experimental import pallas as pl
from jax.experimental.pallas import tpu as pltpu
from jax.experimental.pallas import tpu_sc as plsc

B = 4096
L = 200
EMB = 64
V = 1000000
VW = V // 2
NW = 32
SCALE = 8.0

CHUNK_T = 128
WCH = CHUNK_T // 2
NFULL = V // CHUNK_T
TAIL_T = V - NFULL * CHUNK_T
BASE_CH = NFULL // NW
EXTRA_W = NFULL - BASE_CH * NW

BBLK = B // NW


def _iota16():
    return jax.lax.iota(jnp.int32, 16)


def _t_body(tableT_hbm, tail_hbm, wide_hbm,
            blk0, blk1, buf0, buf1, si0, si1, so0, so1):
    wid = lax.axis_index("s") * 2 + lax.axis_index("c")
    nch = BASE_CH + jnp.where(wid < EXTRA_W, 1, 0)
    lanes = _iota16()
    blks = (blk0, blk1)
    bufs = (buf0, buf1)
    sis = (si0, si1)
    sos = (so0, so1)

    def load(i, slot, sem):
        t0 = pl.multiple_of((wid + NW * i) * CHUNK_T, CHUNK_T)
        pltpu.async_copy(tableT_hbm.at[:, pl.ds(t0, CHUNK_T)],
                         slot.at[:, pl.ds(0, CHUNK_T)], sem)

    def wait_load(slot, sem):
        pltpu.make_async_copy(tableT_hbm.at[:, pl.ds(0, CHUNK_T)],
                              slot.at[:, pl.ds(0, CHUNK_T)], sem).wait()

    def wait_store(slot, sem):
        pltpu.make_async_copy(slot, wide_hbm.at[pl.ds(0, WCH)], sem).wait()

    def transpose(blk, buf):
        jvs = [lanes + 16 * k for k in range(4)]

        def qgroup(q2, carry):
            for qq in range(2):
                q = 2 * q2 + qq
                c0 = jnp.broadcast_to(2 * q, (16,))
                c1 = jnp.broadcast_to(2 * q + 1, (16,))
                vals = [plsc.load_gather(blk, [jvs[k % 4],
                                               c1 if k >= 4 else c0])
                        for k in range(8)]
                for k in range(8):
                    buf[q, pl.ds(16 * k, 16)] = vals[k]
            return carry
        lax.fori_loop(0, WCH // 2, qgroup, 0)

    def store(i, slot, sem):
        w0 = pl.multiple_of((wid + NW * i) * WCH, WCH)
        pltpu.async_copy(slot, wide_hbm.at[pl.ds(w0, WCH)], sem)

    load(0, blk0, si0)

    def outer(i2, carry):
        for b in range(2):
            i = 2 * i2 + b

            @pl.when(i + 1 < nch)
            def _pf():
                load(i + 1, blks[1 - b], sis[1 - b])

            wait_load(blks[b], sis[b])

            @pl.when(i >= 2)
            def _wd():
                wait_store(bufs[b], sos[b])

            transpose(blks[b], bufs[b])
            store(i, bufs[b], sos[b])
        return carry

    lax.fori_loop(0, BASE_CH // 2, outer, 0)

    @pl.when(nch % 2 == 1)
    def _last():
        i = nch - 1
        wait_load(blks[0], sis[0])
        wait_store(bufs[0], sos[0])
        transpose(blks[0], bufs[0])
        store(i, bufs[0], sos[0])

    @pl.when(wid == NW - 1)
    def _tail():
        pltpu.sync_copy(tail_hbm,
                        blk0.at[pl.ds(0, TAIL_T // 2), pl.ds(0, 128)])
        pltpu.sync_copy(blk0.at[pl.ds(0, TAIL_T // 2), pl.ds(0, 128)],
                        wide_hbm.at[pl.ds(VW - TAIL_T // 2, TAIL_T // 2)])

    for b in range(2):
        wait_store(bufs[b], sos[b])


def _g_body(tokT_hbm, wide_hbm, out_hbm, tok_v,
            idx0, idx1, h0, h1, rows0, rows1, buf0, buf1,
            st, sg0, sg1, sw0, sw1):
    wid = lax.axis_index("s") * 2 + lax.axis_index("c")
    b0 = pl.multiple_of(wid * BBLK, BBLK)
    lanes = _iota16()
    idxs = (idx0, idx1)
    hs = (h0, h1)
    rows = (rows0, rows1)
    bufs = (buf0, buf1)
    sgs = (sg0, sg1)
    sws = (sw0, sw1)

    pltpu.async_copy(tokT_hbm.at[:, pl.ds(b0, BBLK)], tok_v, st).wait()

    def prep_and_gather(l, slot):
        for k in range(BBLK // 16):
            t = tok_v[l, pl.ds(16 * k, 16)]
            idxs[slot][pl.ds(16 * k, 16)] = lax.shift_right_logical(t, 1)
            hs[slot][pl.ds(16 * k, 16)] = lax.shift_left(t & 1, 6)
        pltpu.async_copy(wide_hbm.at[idxs[slot]],
                         rows[slot].at[:, pl.ds(0, 128)], sgs[slot])

    def wait_gather(slot):
        pltpu.make_async_copy(wide_hbm.at[idxs[slot]],
                              rows[slot].at[:, pl.ds(0, 128)],
                              sgs[slot]).wait()

    def wait_store(slot):
        pltpu.make_async_copy(bufs[slot],
                              out_hbm.at[0, :, pl.ds(0, BBLK)],
                              sws[slot]).wait()

    def transpose(slot):
        rv = rows[slot]
        bv = bufs[slot]
        for m in range(BBLK // 16):
            rvec = lanes + 16 * m
            hm = hs[slot][pl.ds(16 * m, 16)]

            def jgroup(j2, carry):
                base = j2 * 8
                vals = [plsc.load_gather(rv, [rvec, hm + (base + jj)]) * SCALE
                        for jj in range(8)]
                for jj in range(8):
                    bv[base + jj, pl.ds(16 * m, 16)] = vals[jj]
                return carry
            lax.fori_loop(0, EMB // 8, jgroup, 0)

    prep_and_gather(0, 0)

    def outer(l2, carry):
        for b in range(2):
            l = 2 * l2 + b

            @pl.when(l + 1 < L)
            def _pf():
                prep_and_gather(l + 1, 1 - b)

            wait_gather(b)

            @pl.when(l >= 2)
            def _wd():
                wait_store(b)

            transpose(b)
            pltpu.async_copy(bufs[b], out_hbm.at[l, :, pl.ds(b0, BBLK)],
                             sws[b])
        return carry

    lax.fori_loop(0, L // 2, outer, 0)
    for b in range(2):
        wait_store(b)


def kernel(tokens, table):
    mesh = plsc.VectorSubcoreMesh(core_axis_name="c", subcore_axis_name="s")
    wide = pl.kernel(
        _t_body,
        out_type=jax.ShapeDtypeStruct((VW, 128), jnp.float32),
        mesh=mesh,
        scratch_types=[
            pltpu.VMEM((EMB, CHUNK_T), jnp.float32),
            pltpu.VMEM((EMB, CHUNK_T), jnp.float32),
            pltpu.VMEM((WCH, 128), jnp.float32),
            pltpu.VMEM((WCH, 128), jnp.float32),
            pltpu.SemaphoreType.DMA,
            pltpu.SemaphoreType.DMA,
            pltpu.SemaphoreType.DMA,
            pltpu.SemaphoreType.DMA,
        ],
        compiler_params=pltpu.CompilerParams(needs_layout_passes=False),
    )(table.T, table[NFULL * CHUNK_T:].reshape(TAIL_T // 2, 128))
    outT = pl.kernel(
        _g_body,
        out_type=jax.ShapeDtypeStruct((L, EMB, B), jnp.float32),
        mesh=mesh,
        scratch_types=[
            pltpu.VMEM((L, BBLK), jnp.int32),
            pltpu.VMEM((BBLK,), jnp.int32),
            pltpu.VMEM((BBLK,), jnp.int32),
            pltpu.VMEM((BBLK,), jnp.int32),
            pltpu.VMEM((BBLK,), jnp.int32),
            pltpu.VMEM((BBLK, 128), jnp.float32),
            pltpu.VMEM((BBLK, 128), jnp.float32),
            pltpu.VMEM((EMB, BBLK), jnp.float32),
            pltpu.VMEM((EMB, BBLK), jnp.float32),
            pltpu.SemaphoreType.DMA,
            pltpu.SemaphoreType.DMA,
            pltpu.SemaphoreType.DMA,
            pltpu.SemaphoreType.DMA,
            pltpu.SemaphoreType.DMA,
        ],
        compiler_params=pltpu.CompilerParams(needs_layout_passes=False),
    )(tokens.T.astype(jnp.int32), wide)
    return outT.transpose(2, 0, 1)


# speedup vs baseline: 1.8420x; 1.1826x over previous
import jax
import jax.numpy as jnp
from jax import lax
from jax.experimental import pallas as pl
from jax.experimental.pallas import tpu as pltpu
from jax.experimental.pallas import tpu_sc as plsc

B = 4096
L = 200
EMB = 64
N = B * L
NW = 32
N_W = N // NW
C = 128
NCHUNK = N_W // C
SCALE = 8.0


def _body(tokens_hbm, table_hbm, out_hbm, idx_v, rows_v, gsem):
    wid = lax.axis_index("s") * 2 + lax.axis_index("c")
    base = wid * N_W

    def chunk(g, carry):
        off = base + g * C
        pltpu.sync_copy(tokens_hbm.at[pl.ds(off, C)], idx_v)
        pltpu.async_copy(table_hbm.at[idx_v], rows_v, gsem).wait()

        def row(r, c2):
            for j in range(EMB // 16):
                rows_v[r, pl.ds(16 * j, 16)] = rows_v[r, pl.ds(16 * j, 16)] * SCALE
            return c2

        lax.fori_loop(0, C, row, 0)
        pltpu.sync_copy(rows_v, out_hbm.at[pl.ds(off, C)])
        return carry

    lax.fori_loop(0, NCHUNK, chunk, 0)


def kernel(tokens, table):
    flat = tokens.reshape(N).astype(jnp.int32)
    mesh = plsc.VectorSubcoreMesh(core_axis_name="c", subcore_axis_name="s")
    out = pl.kernel(
        _body,
        out_type=jax.ShapeDtypeStruct((N, EMB), jnp.float32),
        mesh=mesh,
        scratch_types=[
            pltpu.VMEM((C,), jnp.int32),
            pltpu.VMEM((C, EMB), jnp.float32),
            pltpu.SemaphoreType.DMA,
        ],
        compiler_params=pltpu.CompilerParams(use_tc_tiling_on_sc=False),
    )(flat, table)
    return out.reshape(B, L, EMB)
